# baseline (speedup 1.0000x reference)
import functools

import jax
import jax.numpy as jnp
from jax import lax
from jax.experimental import pallas as pl
from jax.experimental.pallas import tpu as pltpu
from jax.experimental.pallas import tpu_sc as plsc

B = 16384
EMB = 32
NC, NS = 2, 16
NW = NC * NS
BPW = B // NW
BLK = 2048

_mesh = plsc.VectorSubcoreMesh(core_axis_name="c", subcore_axis_name="s")


@functools.partial(
    pl.kernel, mesh=_mesh,
    out_type=[jax.ShapeDtypeStruct((B, EMB), jnp.float32)] * 5,
    compiler_params=pltpu.CompilerParams(use_tc_tiling_on_sc=False),
    scratch_types=[
        pltpu.VMEM((BPW,), jnp.int32),
        pltpu.VMEM((BPW, EMB), jnp.float32),
        pltpu.SemaphoreType.DMA,
    ],
)
def _sc_gather(si, gi, oi, ui, mi, st, gt, ot, ut, mt,
               o0, o1, o2, o3, o4, idx_v, rows_v, sem):
    wid = lax.axis_index("s") * NC + lax.axis_index("c")
    base = wid * BPW
    for idx_hbm, tab_hbm, out_hbm in ((si, st, o0), (gi, gt, o1),
                                      (oi, ot, o2), (ui, ut, o3),
                                      (mi, mt, o4)):
        pltpu.sync_copy(idx_hbm.at[pl.ds(base, BPW)], idx_v)
        pltpu.async_copy(tab_hbm.at[idx_v], rows_v, sem).wait()
        pltpu.sync_copy(rows_v, out_hbm.at[pl.ds(base, BPW)])


def _mlp_body(e0_ref, e1_ref, e2_ref, e3_ref, e4_ref,
              w1_ref, b1_ref, w2_ref, b2_ref, w3_ref, b3_ref, out_ref):
    ec = jnp.concatenate([e0_ref[...], e1_ref[...], e2_ref[...],
                          e3_ref[...], e4_ref[...]], axis=1)
    h1 = jnp.maximum(
        jnp.dot(ec, w1_ref[...], preferred_element_type=jnp.float32)
        + b1_ref[...], 0.0)
    h2 = jnp.maximum(
        jnp.dot(h1, w2_ref[...], preferred_element_type=jnp.float32)
        + b2_ref[...], 0.0)
    out_ref[...] = (jnp.dot(h2, w3_ref[...], preferred_element_type=jnp.float32)
                    + b3_ref[...])


@jax.jit
def kernel(school_idx, grade_idx, goal_idx, subject_idx, method_idx,
           school_table, grade_table, goal_table, subject_table, method_table,
           W1, b1, W2, b2, W3, b3):
    idxs = [i.astype(jnp.int32)
            for i in (school_idx, grade_idx, goal_idx, subject_idx,
                      method_idx)]
    e = _sc_gather(*idxs, school_table, grade_table, goal_table,
                   subject_table, method_table)

    grid = B // BLK
    eb_spec = pl.BlockSpec((BLK, EMB), lambda i: (i, 0))
    full = lambda s: pl.BlockSpec(s, lambda i: tuple(0 for _ in s))
    out = pl.pallas_call(
        _mlp_body,
        grid=(grid,),
        in_specs=[eb_spec] * 5 + [
            full((5 * EMB, 256)), full((1, 256)),
            full((256, 128)), full((1, 128)),
            full((128, 32)), full((1, 32)),
        ],
        out_specs=pl.BlockSpec((BLK, 32), lambda i: (i, 0)),
        out_shape=jax.ShapeDtypeStruct((B, 32), jnp.float32),
        compiler_params=pltpu.CompilerParams(
            dimension_semantics=("arbitrary",)),
    )(*e, W1, b1.reshape(1, 256), W2, b2.reshape(1, 128),
      W3, b3.reshape(1, 32))
    return out


# speedup vs baseline: 1.1621x; 1.1621x over previous
import functools

import jax
import jax.numpy as jnp
from jax import lax
from jax.experimental import pallas as pl
from jax.experimental.pallas import tpu as pltpu
from jax.experimental.pallas import tpu_sc as plsc

B = 16384
EMB = 32
NC, NS = 2, 16
NW = NC * NS
BPW = B // NW
BLK = 2048

_mesh = plsc.VectorSubcoreMesh(core_axis_name="c", subcore_axis_name="s")


@functools.partial(
    pl.kernel, mesh=_mesh,
    out_type=[jax.ShapeDtypeStruct((B, EMB), jnp.float32)] * 5,
    compiler_params=pltpu.CompilerParams(use_tc_tiling_on_sc=False),
    scratch_types=(
        [pltpu.VMEM((BPW,), jnp.int32)] * 5
        + [pltpu.VMEM((BPW, EMB), jnp.float32)] * 5
        + [pltpu.SemaphoreType.DMA] * 3
    ),
)
def _sc_gather(si, gi, oi, ui, mi, st, gt, ot, ut, mt,
               o0, o1, o2, o3, o4,
               i0, i1, i2, i3, i4, r0, r1, r2, r3, r4,
               sem_i, sem_g, sem_s):
    wid = lax.axis_index("s") * NC + lax.axis_index("c")
    base = wid * BPW
    idxs = (si, gi, oi, ui, mi)
    tabs = (st, gt, ot, ut, mt)
    outs = (o0, o1, o2, o3, o4)
    ivs = (i0, i1, i2, i3, i4)
    rvs = (r0, r1, r2, r3, r4)
    icps = [pltpu.async_copy(idxs[f].at[pl.ds(base, BPW)], ivs[f], sem_i)
            for f in range(5)]
    gcps = []
    for f in range(5):
        icps[f].wait()
        gcps.append(pltpu.async_copy(tabs[f].at[ivs[f]], rvs[f], sem_g))
    scps = []
    for f in range(5):
        gcps[f].wait()
        scps.append(pltpu.async_copy(rvs[f], outs[f].at[pl.ds(base, BPW)],
                                     sem_s))
    for f in range(5):
        scps[f].wait()


def _mlp_body(e0_ref, e1_ref, e2_ref, e3_ref, e4_ref,
              w1_ref, b1_ref, w2_ref, b2_ref, w3_ref, b3_ref, out_ref):
    ec = jnp.concatenate([e0_ref[...], e1_ref[...], e2_ref[...],
                          e3_ref[...], e4_ref[...]], axis=1)
    h1 = jnp.maximum(
        jnp.dot(ec, w1_ref[...], preferred_element_type=jnp.float32)
        + b1_ref[...], 0.0)
    h2 = jnp.maximum(
        jnp.dot(h1, w2_ref[...], preferred_element_type=jnp.float32)
        + b2_ref[...], 0.0)
    out_ref[...] = (jnp.dot(h2, w3_ref[...], preferred_element_type=jnp.float32)
                    + b3_ref[...])


@jax.jit
def kernel(school_idx, grade_idx, goal_idx, subject_idx, method_idx,
           school_table, grade_table, goal_table, subject_table, method_table,
           W1, b1, W2, b2, W3, b3):
    idxs = [i.astype(jnp.int32)
            for i in (school_idx, grade_idx, goal_idx, subject_idx,
                      method_idx)]
    e = _sc_gather(*idxs, school_table, grade_table, goal_table,
                   subject_table, method_table)

    grid = B // BLK
    eb_spec = pl.BlockSpec((BLK, EMB), lambda i: (i, 0))
    full = lambda s: pl.BlockSpec(s, lambda i: tuple(0 for _ in s))
    out = pl.pallas_call(
        _mlp_body,
        grid=(grid,),
        in_specs=[eb_spec] * 5 + [
            full((5 * EMB, 256)), full((1, 256)),
            full((256, 128)), full((1, 128)),
            full((128, 32)), full((1, 32)),
        ],
        out_specs=pl.BlockSpec((BLK, 32), lambda i: (i, 0)),
        out_shape=jax.ShapeDtypeStruct((B, 32), jnp.float32),
        compiler_params=pltpu.CompilerParams(
            dimension_semantics=("arbitrary",)),
    )(*e, W1, b1.reshape(1, 256), W2, b2.reshape(1, 128),
      W3, b3.reshape(1, 32))
    return out


# speedup vs baseline: 12.2430x; 10.5354x over previous
import functools

import jax
import jax.numpy as jnp
from jax.experimental import pallas as pl
from jax.experimental.pallas import tpu as pltpu

B = 16384
EMB = 32
VSIZES = (52, 14, 12, 14, 8)
PV = (56, 16, 16, 16, 8)
POFF = (0, 56, 72, 88, 104)
VPAD = 128
BLK = 4096


def _body(si_ref, gi_ref, oi_ref, ui_ref, mi_ref,
          st_ref, gt_ref, ot_ref, ut_ref, mt_ref,
          w1_ref, b1_ref, w2_ref, b2_ref, w3_ref, b3_ref,
          out_ref, m_ref):
    @pl.when(pl.program_id(0) == 0)
    def _fold():
        m_ref[...] = jnp.zeros((VPAD, 256), jnp.float32)
        for f, t_ref in enumerate((st_ref, gt_ref, ot_ref, ut_ref, mt_ref)):
            t = t_ref[...]
            if PV[f] > VSIZES[f]:
                t = jnp.concatenate(
                    [t, jnp.zeros((PV[f] - VSIZES[f], EMB), jnp.float32)], 0)
            w1f = w1_ref[f * EMB:(f + 1) * EMB, :]
            m_ref[POFF[f]:POFF[f] + PV[f], :] = jnp.dot(
                t, w1f, preferred_element_type=jnp.float32)

    iota = jax.lax.broadcasted_iota(jnp.int32, (VPAD, BLK), 0)
    acc = None
    for f, i_ref in enumerate((si_ref, gi_ref, oi_ref, ui_ref, mi_ref)):
        hot = (iota == i_ref[0] + POFF[f])
        acc = hot if acc is None else jnp.logical_or(acc, hot)
    a_t = acc.astype(jnp.float32)

    h1 = jnp.maximum(
        jax.lax.dot_general(a_t, m_ref[...], (((0,), (0,)), ((), ())),
                            preferred_element_type=jnp.float32)
        + b1_ref[...], 0.0)
    h2 = jnp.maximum(
        jnp.dot(h1, w2_ref[...], preferred_element_type=jnp.float32)
        + b2_ref[...], 0.0)
    out_ref[...] = (jnp.dot(h2, w3_ref[...], preferred_element_type=jnp.float32)
                    + b3_ref[...])


@jax.jit
def kernel(school_idx, grade_idx, goal_idx, subject_idx, method_idx,
           school_table, grade_table, goal_table, subject_table, method_table,
           W1, b1, W2, b2, W3, b3):
    grid = B // BLK
    idxs = [i.astype(jnp.int32).reshape(grid, 1, BLK)
            for i in (school_idx, grade_idx, goal_idx, subject_idx,
                      method_idx)]
    idx_spec = pl.BlockSpec((1, 1, BLK), lambda i: (i, 0, 0))
    full = lambda s: pl.BlockSpec(s, lambda i: tuple(0 for _ in s))
    out = pl.pallas_call(
        _body,
        grid=(grid,),
        in_specs=[idx_spec] * 5 + [
            full((VSIZES[0], EMB)), full((VSIZES[1], EMB)),
            full((VSIZES[2], EMB)), full((VSIZES[3], EMB)),
            full((VSIZES[4], EMB)),
            full((5 * EMB, 256)), full((1, 256)),
            full((256, 128)), full((1, 128)),
            full((128, 32)), full((1, 32)),
        ],
        out_specs=pl.BlockSpec((BLK, 32), lambda i: (i, 0)),
        out_shape=jax.ShapeDtypeStruct((B, 32), jnp.float32),
        scratch_shapes=[pltpu.VMEM((VPAD, 256), jnp.float32)],
        compiler_params=pltpu.CompilerParams(
            dimension_semantics=("arbitrary",)),
    )(*idxs, school_table, grade_table, goal_table, subject_table,
      method_table, W1, b1.reshape(1, 256), W2, b2.reshape(1, 128),
      W3, b3.reshape(1, 32))
    return out
